# baseline (speedup 1.0000x reference)
import numpy as np
import jax
import jax.numpy as jnp
from jax import lax
from jax.experimental import pallas as pl
from jax.experimental.pallas import tpu as pltpu

_EPS = 0.1
_N = 16384
_C = 1000
_CB = 1024
_G = _N // (2 * _CB)
_T1 = np.float32((1.0 - _EPS) ** 2 * _N)
_T2 = np.float32(1.0 - _EPS)
_INF_BITS = np.int32(0x7F800000)


def _loss_margin(x, tgt):
    m1 = jnp.max(x, axis=0, keepdims=True)
    eq = x == m1
    dup = jnp.sum(eq.astype(jnp.float32), axis=0, keepdims=True) > 1.0
    m2c = jnp.max(jnp.where(eq, -jnp.inf, x), axis=0, keepdims=True)
    m2 = jnp.where(dup, m1, m2c)
    rows = lax.broadcasted_iota(jnp.int32, x.shape, 0).astype(jnp.float32)
    tsc = jnp.sum(jnp.where(rows == tgt, x, 0.0), axis=0, keepdims=True)
    se = jnp.sum(jnp.exp(x - m1), axis=0, keepdims=True)
    lse = m1 + jnp.log(se)
    margin = tsc - m2
    fst = jnp.maximum(1.0 - margin, 0.0)
    snd = jnp.maximum(1.0 - tsc + lse, 0.0)
    lossv = jnp.where(margin >= 0.0, fst, snd)
    return lossv, margin


def _npc_kernel(xa_ref, xb_ref, ta_ref, tb_ref, out_ref, loss_s, margin_s):
    i = pl.program_id(0)
    la, ma = _loss_margin(xa_ref[...], ta_ref[...])
    lb, mb = _loss_margin(xb_ref[...], tb_ref[...])
    loss_s[pl.ds(2 * i, 1), :] = la
    loss_s[pl.ds(2 * i + 1, 1), :] = lb
    margin_s[pl.ds(2 * i, 1), :] = ma
    margin_s[pl.ds(2 * i + 1, 1), :] = mb

    @pl.when(i == _G - 1)
    def _select():
        loss = loss_s[...]
        marg = margin_s[...]
        nf = np.float32(_N)
        cnt = jnp.sum((marg < 0.0).astype(jnp.float32))
        threshold = _T1 + _T2 * cnt
        t1 = threshold + np.float32(1.0)

        def body(_, carry):
            lo, hi = carry
            mid = lo + lax.div(hi - lo + 1, 2)
            v = lax.bitcast_convert_type(mid, jnp.float32)
            g = jnp.sum(jnp.where(loss <= v, loss + 1.0, 0.0))
            ok = g <= t1
            return (jnp.where(ok, mid, lo), jnp.where(ok, hi, mid - 1))

        lo, _hi = lax.fori_loop(0, 31, body, (jnp.int32(-1), _INF_BITS))
        none = lo < 0
        v0 = lax.bitcast_convert_type(jnp.maximum(lo, 0), jnp.float32)
        m0 = jnp.logical_and(loss <= v0, jnp.logical_not(none))
        c0 = jnp.sum(jnp.where(m0, 1.0, 0.0))
        s0 = jnp.sum(jnp.where(m0, loss, 0.0))
        bi = (loss + np.float32(1e-10)) / loss
        sb0 = jnp.sum(jnp.where(m0, bi, 0.0))
        vn = jnp.min(jnp.where(m0, jnp.inf, loss))
        mn = jnp.sum(jnp.where(loss == vn, 1.0, 0.0))
        traw = jnp.floor((threshold - s0 - c0 - vn) / (vn + 1.0)) + 1.0
        t = jnp.clip(traw, 0.0, mn)
        t = jnp.where(c0 >= nf, 0.0, t)
        npcl1 = s0 + jnp.where(t > 0.0, t * vn, 0.0)
        bin_ = (vn + np.float32(1e-10)) / vn
        npcl2 = threshold - (sb0 + jnp.where(t > 0.0, t * bin_, 0.0))
        res = jnp.maximum(npcl1, npcl2) / nf * np.float32(0.1)
        out_ref[...] = jnp.broadcast_to(res, (1, 1))


def kernel(output, target):
    xt = output.T
    tgt2d = target.astype(jnp.float32).reshape(1, _N)
    out = pl.pallas_call(
        _npc_kernel,
        grid=(_G,),
        in_specs=[
            pl.BlockSpec((_C, _CB), lambda i: (0, 2 * i)),
            pl.BlockSpec((_C, _CB), lambda i: (0, 2 * i + 1)),
            pl.BlockSpec((1, _CB), lambda i: (0, 2 * i)),
            pl.BlockSpec((1, _CB), lambda i: (0, 2 * i + 1)),
        ],
        out_specs=pl.BlockSpec((1, 1), lambda i: (0, 0)),
        out_shape=jax.ShapeDtypeStruct((1, 1), jnp.float32),
        scratch_shapes=[
            pltpu.VMEM((2 * _G, _CB), jnp.float32),
            pltpu.VMEM((2 * _G, _CB), jnp.float32),
        ],
    )(xt, xt, tgt2d, tgt2d)
    return out[0, 0]


# speedup vs baseline: 1.0233x; 1.0233x over previous
import numpy as np
import jax
import jax.numpy as jnp
from jax import lax
from jax.experimental import pallas as pl
from jax.experimental.pallas import tpu as pltpu

_EPS = 0.1
_N = 16384
_C = 1000
_CB = 2048
_G = _N // _CB
_T1 = np.float32((1.0 - _EPS) ** 2 * _N)
_T2 = np.float32(1.0 - _EPS)
_INF_BITS = np.int32(0x7F800000)


def _npc_kernel(x_ref, t_ref, out_ref, loss_s, margin_s):
    i = pl.program_id(0)
    x = x_ref[...]
    tgt = t_ref[...]
    m1 = jnp.max(x, axis=0, keepdims=True)
    eq = x == m1
    dup = jnp.sum(eq.astype(jnp.float32), axis=0, keepdims=True) > 1.0
    m2c = jnp.max(jnp.where(eq, -jnp.inf, x), axis=0, keepdims=True)
    m2 = jnp.where(dup, m1, m2c)
    rows = lax.broadcasted_iota(jnp.int32, x.shape, 0).astype(jnp.float32)
    tsc = jnp.sum(jnp.where(rows == tgt, x, 0.0), axis=0, keepdims=True)
    se = jnp.sum(jnp.exp(x - m1), axis=0, keepdims=True)
    lse = m1 + jnp.log(se)
    margin = tsc - m2
    fst = jnp.maximum(1.0 - margin, 0.0)
    snd = jnp.maximum(1.0 - tsc + lse, 0.0)
    lossv = jnp.where(margin >= 0.0, fst, snd)
    loss_s[pl.ds(i, 1), :] = lossv
    margin_s[pl.ds(i, 1), :] = margin

    @pl.when(i == _G - 1)
    def _select():
        loss = loss_s[...]
        marg = margin_s[...]
        nf = np.float32(_N)
        cnt = jnp.sum((marg < 0.0).astype(jnp.float32))
        threshold = _T1 + _T2 * cnt
        t1 = threshold + np.float32(1.0)
        lp1 = loss + 1.0

        def body(_, carry):
            lo, hi = carry
            mid = lo + lax.div(hi - lo + 1, 2)
            v = lax.bitcast_convert_type(mid, jnp.float32)
            g = jnp.sum(jnp.where(loss <= v, lp1, 0.0))
            ok = g <= t1
            return (jnp.where(ok, mid, lo), jnp.where(ok, hi, mid - 1))

        lo, _hi = lax.fori_loop(0, 31, body, (jnp.int32(-1), _INF_BITS))
        none = lo < 0
        v0 = lax.bitcast_convert_type(jnp.maximum(lo, 0), jnp.float32)
        m0 = jnp.logical_and(loss <= v0, jnp.logical_not(none))
        c0 = jnp.sum(jnp.where(m0, 1.0, 0.0))
        s0 = jnp.sum(jnp.where(m0, loss, 0.0))
        bi = (loss + np.float32(1e-10)) / loss
        sb0 = jnp.sum(jnp.where(m0, bi, 0.0))
        vn = jnp.min(jnp.where(m0, jnp.inf, loss))
        mn = jnp.sum(jnp.where(loss == vn, 1.0, 0.0))
        traw = jnp.floor((threshold - s0 - c0 - vn) / (vn + 1.0)) + 1.0
        t = jnp.clip(traw, 0.0, mn)
        t = jnp.where(c0 >= nf, 0.0, t)
        npcl1 = s0 + jnp.where(t > 0.0, t * vn, 0.0)
        bin_ = (vn + np.float32(1e-10)) / vn
        npcl2 = threshold - (sb0 + jnp.where(t > 0.0, t * bin_, 0.0))
        res = jnp.maximum(npcl1, npcl2) / nf * np.float32(0.1)
        out_ref[...] = jnp.broadcast_to(res, (1, 1))


def kernel(output, target):
    xt = output.T
    tgt2d = target.astype(jnp.float32).reshape(1, _N)
    out = pl.pallas_call(
        _npc_kernel,
        grid=(_G,),
        in_specs=[
            pl.BlockSpec((_C, _CB), lambda i: (0, i)),
            pl.BlockSpec((1, _CB), lambda i: (0, i)),
        ],
        out_specs=pl.BlockSpec((1, 1), lambda i: (0, 0)),
        out_shape=jax.ShapeDtypeStruct((1, 1), jnp.float32),
        scratch_shapes=[
            pltpu.VMEM((_G, _CB), jnp.float32),
            pltpu.VMEM((_G, _CB), jnp.float32),
        ],
    )(xt, tgt2d)
    return out[0, 0]
